# baseline (speedup 1.0000x reference)
import functools

import jax
import jax.numpy as jnp
from jax import lax
from jax.experimental import pallas as pl
from jax.experimental.pallas import tpu as pltpu
from jax.experimental.pallas import tpu_sc as plsc

N = 16384
EMB = 33
D_HALF = 32
V = 100000

NUM_CORES = 2
NUM_SUBCORES = 16
NW = NUM_CORES * NUM_SUBCORES
L = 16
CHUNK = 2048
N_CHUNKS = N // CHUNK


_sc_mesh = plsc.VectorSubcoreMesh(core_axis_name="c", subcore_axis_name="s")


@functools.partial(
    pl.kernel,
    mesh=_sc_mesh,
    out_type=jax.ShapeDtypeStruct((D_HALF, N), jnp.float32),
    compiler_params=pltpu.CompilerParams(needs_layout_passes=False),
    scratch_types=[
        pltpu.VMEM((1, V), jnp.float32),
        pltpu.VMEM((N,), jnp.int32),
        pltpu.VMEM((2, CHUNK), jnp.float32),
        pltpu.SemaphoreType.DMA,
        pltpu.SemaphoreType.DMA,
    ],
)
def _gather_sc(tableT_hbm, idx_hbm, embT_hbm, row_v, idx_v, out_v, sem, osem):
    wid = lax.axis_index("s") * NUM_CORES + lax.axis_index("c")
    row_cp = pltpu.async_copy(tableT_hbm.at[pl.ds(wid, 1), :], row_v, sem)
    idx_cp = pltpu.async_copy(idx_hbm, idx_v, sem)
    row_cp.wait()
    idx_cp.wait()
    zeros = lax.iota(jnp.int32, L) * 0
    UNROLL = 16

    out_cps = [None, None]
    for c in range(N_CHUNKS):
        buf = c % 2

        def group_body(gg, _, c=c, buf=buf):
            base = gg * (L * UNROLL)
            for u in range(UNROLL):
                off = base + u * L
                iv = idx_v[pl.ds(c * CHUNK + off, L)]
                out_v[buf, pl.ds(off, L)] = plsc.load_gather(
                    row_v, [zeros, iv]
                )
            return 0

        if out_cps[buf] is not None:
            out_cps[buf].wait()
        lax.fori_loop(0, CHUNK // (L * UNROLL), group_body, 0)
        out_cps[buf] = pltpu.async_copy(
            out_v.at[pl.ds(buf, 1)],
            embT_hbm.at[pl.ds(wid, 1), pl.ds(c * CHUNK, CHUNK)],
            osem,
        )
    for cp in out_cps:
        cp.wait()


def _fuse_body(tokT_ref, embT_ref, w_ref, b_ref, out_ref):
    x = tokT_ref[1:EMB, :]
    y = jnp.dot(w_ref[:], x, preferred_element_type=jnp.float32)
    proj = jnp.maximum(y + b_ref[:], 0.0)
    out_ref[:] = jnp.concatenate([embT_ref[:], proj], axis=0)


_BLK = 2048


def _fuse_tc(tokenT, embT, W, b):
    b2 = b.reshape(D_HALF, 1)
    return pl.pallas_call(
        _fuse_body,
        grid=(N // _BLK,),
        in_specs=[
            pl.BlockSpec((EMB, _BLK), lambda i: (0, i)),
            pl.BlockSpec((D_HALF, _BLK), lambda i: (0, i)),
            pl.BlockSpec((D_HALF, EMB - 1), lambda i: (0, 0)),
            pl.BlockSpec((D_HALF, 1), lambda i: (0, 0)),
        ],
        out_specs=pl.BlockSpec((2 * D_HALF, _BLK), lambda i: (0, i)),
        out_shape=jax.ShapeDtypeStruct((2 * D_HALF, N), jnp.float32),
    )(tokenT, embT, W, b2)


def kernel(token, table, W, b):
    tokenT = token.T
    tableT = table.T
    idx = tokenT[0, :].astype(jnp.int32)
    embT = _gather_sc(tableT, idx)
    outT = _fuse_tc(tokenT, embT, W, b)
    return outT.T


# speedup vs baseline: 1.0068x; 1.0068x over previous
import functools

import jax
import jax.numpy as jnp
from jax import lax
from jax.experimental import pallas as pl
from jax.experimental.pallas import tpu as pltpu
from jax.experimental.pallas import tpu_sc as plsc

N = 16384
EMB = 33
D_HALF = 32
V = 100000

NUM_CORES = 2
NUM_SUBCORES = 16
NW = NUM_CORES * NUM_SUBCORES
L = 16
CHUNK = 2048
N_CHUNKS = N // CHUNK


_sc_mesh = plsc.VectorSubcoreMesh(core_axis_name="c", subcore_axis_name="s")


@functools.partial(
    pl.kernel,
    mesh=_sc_mesh,
    out_type=jax.ShapeDtypeStruct((D_HALF, N), jnp.float32),
    compiler_params=pltpu.CompilerParams(needs_layout_passes=False),
    scratch_types=[
        pltpu.VMEM((1, V), jnp.float32),
        pltpu.VMEM((N,), jnp.int32),
        pltpu.VMEM((2, CHUNK), jnp.float32),
        pltpu.SemaphoreType.DMA,
        pltpu.SemaphoreType.DMA,
    ],
)
def _gather_sc(tableT_hbm, idx_hbm, embT_hbm, row_v, idx_v, out_v, sem, osem):
    wid = lax.axis_index("s") * NUM_CORES + lax.axis_index("c")
    row_cp = pltpu.async_copy(tableT_hbm.at[pl.ds(wid, 1), :], row_v, sem)
    idx_cp = pltpu.async_copy(idx_hbm, idx_v, sem)
    row_cp.wait()
    idx_cp.wait()
    zeros = lax.iota(jnp.int32, L) * 0
    UNROLL = 8

    out_cps = [None, None]
    for c in range(N_CHUNKS):
        buf = c % 2

        def group_body(gg, _, c=c, buf=buf):
            base = gg * (L * UNROLL)
            for u in range(UNROLL):
                off = base + u * L
                iv = idx_v[pl.ds(c * CHUNK + off, L)]
                out_v[buf, pl.ds(off, L)] = plsc.load_gather(
                    row_v, [zeros, iv]
                )
            return 0

        if out_cps[buf] is not None:
            out_cps[buf].wait()
        lax.fori_loop(0, CHUNK // (L * UNROLL), group_body, 0)
        out_cps[buf] = pltpu.async_copy(
            out_v.at[pl.ds(buf, 1)],
            embT_hbm.at[pl.ds(wid, 1), pl.ds(c * CHUNK, CHUNK)],
            osem,
        )
    for cp in out_cps:
        cp.wait()


def _fuse_body(tokT_ref, embT_ref, w_ref, b_ref, out_ref):
    x = tokT_ref[1:EMB, :]
    y = jnp.dot(w_ref[:], x, preferred_element_type=jnp.float32)
    proj = jnp.maximum(y + b_ref[:], 0.0)
    out_ref[:] = jnp.concatenate([embT_ref[:], proj], axis=0)


_BLK = 2048


def _fuse_tc(tokenT, embT, W, b):
    b2 = b.reshape(D_HALF, 1)
    return pl.pallas_call(
        _fuse_body,
        grid=(N // _BLK,),
        in_specs=[
            pl.BlockSpec((EMB, _BLK), lambda i: (0, i)),
            pl.BlockSpec((D_HALF, _BLK), lambda i: (0, i)),
            pl.BlockSpec((D_HALF, EMB - 1), lambda i: (0, 0)),
            pl.BlockSpec((D_HALF, 1), lambda i: (0, 0)),
        ],
        out_specs=pl.BlockSpec((2 * D_HALF, _BLK), lambda i: (0, i)),
        out_shape=jax.ShapeDtypeStruct((2 * D_HALF, N), jnp.float32),
    )(tokenT, embT, W, b2)


def kernel(token, table, W, b):
    tokenT = token.T
    tableT = table.T
    idx = tokenT[0, :].astype(jnp.int32)
    embT = _gather_sc(tableT, idx)
    outT = _fuse_tc(tokenT, embT, W, b)
    return outT.T
